# baseline (speedup 1.0000x reference)
import functools

import jax
import jax.numpy as jnp
from jax import lax
from jax.experimental import pallas as pl
from jax.experimental.pallas import tpu as pltpu
from jax.experimental.pallas import tpu_sc as plsc

CAPACITY = 64
NUM_CLUSTERS = 64
REG = 0.1

_DN_STD = (((1,), (0,)), ((), ()))
_DN1 = (((1,), (1,)), ((), ()))


_CBLK = 512


def _centers_g_body(table_ref, labels_ref, g_ref, sums_ref, counts_ref,
                    centers_ref, csq_ref):
    p = pl.program_id(0)
    i = pl.program_id(1)
    nblk = pl.num_programs(1)
    dim = table_ref.shape[1]
    table = table_ref[...]
    lab_row = labels_ref[0:1, pl.ds(i * _CBLK, _CBLK)]
    k_iota = lax.broadcasted_iota(jnp.int32, (NUM_CLUSTERS, _CBLK), 0)

    @pl.when(p == 0)
    def _phase0():
        onehot_t = (lab_row == k_iota).astype(jnp.float32)
        psum = lax.dot_general(onehot_t, table, _DN_STD,
                               preferred_element_type=jnp.float32,
                               precision=lax.Precision.HIGHEST)
        pcnt = jnp.sum(onehot_t, axis=1, keepdims=True)

        @pl.when(i == 0)
        def _init():
            sums_ref[...] = psum
            counts_ref[...] = pcnt

        @pl.when(i > 0)
        def _acc():
            sums_ref[...] += psum
            counts_ref[...] += pcnt

        @pl.when(i == nblk - 1)
        def _finalize():
            centers = sums_ref[...] / jnp.maximum(counts_ref[...], 1.0)
            centers_ref[...] = centers
            csq_ref[...] = lax.dot_general(
                jnp.ones((1, dim), jnp.float32), centers * centers, _DN1,
                preferred_element_type=jnp.float32,
                precision=lax.Precision.HIGHEST)

    @pl.when(p == 1)
    def _phase1():
        centers = centers_ref[...]
        m = lax.dot_general(table, centers, _DN1,
                            preferred_element_type=jnp.float32,
                            precision=lax.Precision.HIGHEST)
        rowsq = jnp.sum(table * table, axis=1, keepdims=True)
        g = rowsq - 2.0 * m + csq_ref[...]
        g_ref[...] = jnp.concatenate(
            [g, jnp.zeros((_CBLK, 128 - NUM_CLUSTERS), jnp.float32)], axis=1)


def _centers_g_tc(table2d, labels_row):
    rows, dim = table2d.shape
    return pl.pallas_call(
        _centers_g_body,
        grid=(2, rows // _CBLK),
        in_specs=[
            pl.BlockSpec((_CBLK, dim), lambda p, i: (i, 0)),
            pl.BlockSpec((1, rows), lambda p, i: (0, 0)),
        ],
        out_specs=pl.BlockSpec((_CBLK, 128), lambda p, i: (i, 0)),
        out_shape=jax.ShapeDtypeStruct((rows, 128), jnp.float32),
        scratch_shapes=[
            pltpu.VMEM((NUM_CLUSTERS, dim), jnp.float32),
            pltpu.VMEM((NUM_CLUSTERS, 1), jnp.float32),
            pltpu.VMEM((NUM_CLUSTERS, dim), jnp.float32),
            pltpu.VMEM((1, NUM_CLUSTERS), jnp.float32),
        ],
    )(table2d, labels_row)


def _gather_d_sc(gflat, labels2d, rows):
    ncores, nsub, lanes = 2, 16, 16
    nw = ncores * nsub
    rows_per_w = rows // nw

    @functools.partial(
        pl.kernel,
        out_type=jax.ShapeDtypeStruct((2, rows), jnp.float32),
        mesh=plsc.VectorSubcoreMesh(core_axis_name="c", subcore_axis_name="s"),
        scratch_types=[
            pltpu.VMEM((rows_per_w,), jnp.int32),
            pltpu.VMEM((rows_per_w,), jnp.int32),
            pltpu.VMEM((rows_per_w,), jnp.float32),
            pltpu.VMEM((rows_per_w,), jnp.float32),
            pltpu.SemaphoreType.DMA,
        ],
    )
    def gather(gflat_hbm, labels_hbm, row_hbm, lab_v, idx_v, d_v, labf_v, sem):
        c = lax.axis_index("c")
        s = lax.axis_index("s")
        wid = s * ncores + c
        row0 = wid * rows_per_w
        pltpu.sync_copy(labels_hbm.at[wid], lab_v)
        iota = lax.iota(jnp.int32, lanes)
        for j in range(rows_per_w // lanes):
            lab16 = lab_v[pl.ds(j * lanes, lanes)]
            idx_v[pl.ds(j * lanes, lanes)] = (
                (row0 + j * lanes + iota) * 128 + lab16)
            labf_v[pl.ds(j * lanes, lanes)] = lab16.astype(jnp.float32)
        pltpu.async_copy(gflat_hbm.at[idx_v], d_v, sem).wait()
        pltpu.sync_copy(d_v, row_hbm.at[0, pl.ds(row0, rows_per_w)])
        pltpu.sync_copy(labf_v, row_hbm.at[1, pl.ds(row0, rows_per_w)])

    return gather(gflat, labels2d)


_BLK = 512


def _ranks_body(dr_ref, soft_ref, ridx_ref, scores_ref):
    i = pl.program_id(0)
    d_all = dr_ref[0:1, :]
    lab_all = dr_ref[1:2, :]
    rows = d_all.shape[1]
    mn = jnp.min(d_all)
    mx = jnp.max(d_all)
    s_all = (d_all - mn) / (mx - mn) + lab_all
    mn2 = jnp.min(s_all)
    mx2 = jnp.max(s_all)
    c = jnp.float32(0.5 / REG)
    z_all = (s_all - mn2) / (mx2 - mn2) * c
    dsl = dr_ref[0:1, pl.ds(i * _BLK, _BLK)]
    lsl = dr_ref[1:2, pl.ds(i * _BLK, _BLK)]
    s_blk = ((dsl - mn) / (mx - mn) + lsl).reshape(_BLK, 1)
    z_blk = ((s_blk - mn2) / (mx2 - mn2)) * c
    th = jnp.tanh(z_blk - z_all)
    tsum = jnp.sum(th, axis=1, keepdims=True)
    soft_ref[...] = 0.5 * tsum + jnp.float32(0.5 * rows + 0.5)
    j_iota = lax.broadcasted_iota(jnp.int32, (_BLK, rows), 1)
    i_idx = i * _BLK + lax.broadcasted_iota(jnp.int32, (_BLK, 1), 0)
    cond = (s_all < s_blk) | ((s_all == s_blk) & (j_iota < i_idx))
    cnt = jnp.sum(cond.astype(jnp.int32), axis=1, keepdims=True)
    ridx_ref[...] = cnt // CAPACITY + 1
    scores_ref[...] = s_blk


def _ranks_tc(drow2):
    rows = drow2.shape[1]
    grid = rows // _BLK
    return pl.pallas_call(
        _ranks_body,
        grid=(grid,),
        in_specs=[
            pl.BlockSpec((2, rows), lambda i: (0, 0)),
        ],
        out_specs=[
            pl.BlockSpec((_BLK, 1), lambda i: (i, 0)),
            pl.BlockSpec((_BLK, 1), lambda i: (i, 0)),
            pl.BlockSpec((_BLK, 1), lambda i: (i, 0)),
        ],
        out_shape=[
            jax.ShapeDtypeStruct((rows, 1), jnp.float32),
            jax.ShapeDtypeStruct((rows, 1), jnp.int32),
            jax.ShapeDtypeStruct((rows, 1), jnp.float32),
        ],
    )(drow2)


def kernel(table, labels):
    rows = table.shape[1]
    dim = table.shape[-1]
    table2d = table.reshape(rows, dim)
    g = _centers_g_tc(table2d, labels.reshape(1, rows))
    drow2 = _gather_d_sc(g.reshape(rows * 128),
                         labels.reshape(rows // 128, 128), rows)
    soft, ridx, scores = _ranks_tc(drow2)
    return (soft.reshape(1, rows, 1),
            ridx.reshape(1, rows, 1),
            scores.reshape(1, rows, 1))


# speedup vs baseline: 1.2156x; 1.2156x over previous
import functools

import jax
import jax.numpy as jnp
from jax import lax
from jax.experimental import pallas as pl
from jax.experimental.pallas import tpu as pltpu
from jax.experimental.pallas import tpu_sc as plsc

CAPACITY = 64
NUM_CLUSTERS = 64
REG = 0.1

_DN_STD = (((1,), (0,)), ((), ()))
_DN1 = (((1,), (1,)), ((), ()))


def _centers_g_body(table_ref, labels_ref, g_ref):
    rows, dim = table_ref.shape
    table = table_ref[...]
    lab_row = labels_ref[...]
    k_iota = lax.broadcasted_iota(jnp.int32, (NUM_CLUSTERS, rows), 0)
    onehot_t = (lab_row == k_iota).astype(jnp.float32)
    sums = lax.dot_general(onehot_t, table, _DN_STD,
                           preferred_element_type=jnp.float32)
    counts = jnp.sum(onehot_t, axis=1, keepdims=True)
    centers = sums / jnp.maximum(counts, 1.0)
    m = lax.dot_general(table, centers, _DN1,
                        preferred_element_type=jnp.float32)
    rowsq = jnp.sum(table * table, axis=1, keepdims=True)
    csq = lax.dot_general(jnp.ones((1, dim), jnp.float32), centers * centers,
                          _DN1,
                          preferred_element_type=jnp.float32,
                          precision=lax.Precision.HIGHEST)
    g = rowsq - 2.0 * m + csq
    g_ref[...] = jnp.concatenate(
        [g, jnp.zeros((rows, 128 - NUM_CLUSTERS), jnp.float32)], axis=1)


def _centers_g_tc(table2d, labels_row):
    rows = table2d.shape[0]
    return pl.pallas_call(
        _centers_g_body,
        out_shape=jax.ShapeDtypeStruct((rows, 128), jnp.float32),
    )(table2d, labels_row)


def _gather_d_sc(gflat, labels2d, rows):
    ncores, nsub, lanes = 2, 16, 16
    nw = ncores * nsub
    rows_per_w = rows // nw

    @functools.partial(
        pl.kernel,
        out_type=jax.ShapeDtypeStruct((2, rows), jnp.float32),
        mesh=plsc.VectorSubcoreMesh(core_axis_name="c", subcore_axis_name="s"),
        scratch_types=[
            pltpu.VMEM((rows_per_w,), jnp.int32),
            pltpu.VMEM((rows_per_w,), jnp.int32),
            pltpu.VMEM((rows_per_w,), jnp.float32),
            pltpu.VMEM((rows_per_w,), jnp.float32),
            pltpu.SemaphoreType.DMA,
        ],
    )
    def gather(gflat_hbm, labels_hbm, row_hbm, lab_v, idx_v, d_v, labf_v, sem):
        c = lax.axis_index("c")
        s = lax.axis_index("s")
        wid = s * ncores + c
        row0 = wid * rows_per_w
        pltpu.sync_copy(labels_hbm.at[wid], lab_v)
        iota = lax.iota(jnp.int32, lanes)
        for j in range(rows_per_w // lanes):
            lab16 = lab_v[pl.ds(j * lanes, lanes)]
            idx_v[pl.ds(j * lanes, lanes)] = (
                (row0 + j * lanes + iota) * 128 + lab16)
            labf_v[pl.ds(j * lanes, lanes)] = lab16.astype(jnp.float32)
        pltpu.async_copy(gflat_hbm.at[idx_v], d_v, sem).wait()
        pltpu.sync_copy(d_v, row_hbm.at[0, pl.ds(row0, rows_per_w)])
        pltpu.sync_copy(labf_v, row_hbm.at[1, pl.ds(row0, rows_per_w)])

    return gather(gflat, labels2d)


_BLK = 512


def _ranks_body(dr_ref, soft_ref, ridx_ref, scores_ref):
    i = pl.program_id(0)
    d_all = dr_ref[0:1, :]
    lab_all = dr_ref[1:2, :]
    rows = d_all.shape[1]
    mn = jnp.min(d_all)
    mx = jnp.max(d_all)
    s_all = (d_all - mn) / (mx - mn) + lab_all
    mn2 = jnp.min(s_all)
    mx2 = jnp.max(s_all)
    c = jnp.float32(0.5 / REG)
    z_all = (s_all - mn2) / (mx2 - mn2) * c
    dsl = dr_ref[0:1, pl.ds(i * _BLK, _BLK)]
    lsl = dr_ref[1:2, pl.ds(i * _BLK, _BLK)]
    s_blk = ((dsl - mn) / (mx - mn) + lsl).reshape(_BLK, 1)
    z_blk = ((s_blk - mn2) / (mx2 - mn2)) * c
    th = jnp.tanh(z_blk - z_all)
    tsum = jnp.sum(th, axis=1, keepdims=True)
    soft_ref[...] = 0.5 * tsum + jnp.float32(0.5 * rows + 0.5)
    j_iota = lax.broadcasted_iota(jnp.int32, (_BLK, rows), 1)
    i_idx = i * _BLK + lax.broadcasted_iota(jnp.int32, (_BLK, 1), 0)
    cond = (s_all < s_blk) | ((s_all == s_blk) & (j_iota < i_idx))
    cnt = jnp.sum(cond.astype(jnp.int32), axis=1, keepdims=True)
    ridx_ref[...] = cnt // CAPACITY + 1
    scores_ref[...] = s_blk


def _ranks_tc(drow2):
    rows = drow2.shape[1]
    grid = rows // _BLK
    return pl.pallas_call(
        _ranks_body,
        grid=(grid,),
        in_specs=[
            pl.BlockSpec((2, rows), lambda i: (0, 0)),
        ],
        out_specs=[
            pl.BlockSpec((_BLK, 1), lambda i: (i, 0)),
            pl.BlockSpec((_BLK, 1), lambda i: (i, 0)),
            pl.BlockSpec((_BLK, 1), lambda i: (i, 0)),
        ],
        out_shape=[
            jax.ShapeDtypeStruct((rows, 1), jnp.float32),
            jax.ShapeDtypeStruct((rows, 1), jnp.int32),
            jax.ShapeDtypeStruct((rows, 1), jnp.float32),
        ],
    )(drow2)


def kernel(table, labels):
    rows = table.shape[1]
    dim = table.shape[-1]
    table2d = table.reshape(rows, dim)
    g = _centers_g_tc(table2d, labels.reshape(1, rows))
    drow2 = _gather_d_sc(g.reshape(rows * 128),
                         labels.reshape(rows // 128, 128), rows)
    soft, ridx, scores = _ranks_tc(drow2)
    return (soft.reshape(1, rows, 1),
            ridx.reshape(1, rows, 1),
            scores.reshape(1, rows, 1))
